# Initial kernel scaffold; baseline (speedup 1.0000x reference)
import functools

import jax
import jax.numpy as jnp
from jax.experimental import pallas as pl
from jax.experimental.pallas import tpu as pltpu
from jax.experimental.pallas import tpu_sc as plsc

NC = 2
NS = 16
LANES = 16
NW = NC * NS


def _sc_aggregate(x, row, col):
    n, d = x.shape
    e = row.shape[0]
    epw = e // NW
    B = 80
    chunks = epw // B
    WB = 80
    nwb = n // WB
    wb_per = (nwb + NS - 1) // NS

    mesh = plsc.VectorSubcoreMesh(
        core_axis_name="c", subcore_axis_name="s",
        num_cores=NC, num_subcores=NS)

    z_feat = jnp.zeros((WB, d), jnp.float32)
    z_deg = jnp.zeros((WB, LANES), jnp.float32)
    ones_rows = jnp.ones((B, LANES), jnp.float32)

    @functools.partial(
        pl.kernel,
        out_type=(jax.ShapeDtypeStruct((NC * n, d), jnp.float32),
                  jax.ShapeDtypeStruct((NC * n, LANES), jnp.float32)),
        mesh=mesh,
        scratch_types=[
            pltpu.VMEM((B,), jnp.int32),
            pltpu.VMEM((B,), jnp.int32),
            pltpu.VMEM((B, d), jnp.float32),
            pltpu.VMEM((B, LANES), jnp.float32),
            pltpu.VMEM_SHARED((n, d), jnp.float32),
            pltpu.VMEM_SHARED((n, LANES), jnp.float32),
        ],
    )
    def agg_kernel(x_hbm, row_hbm, col_hbm, zf_hbm, zd_hbm, ones_hbm,
                   sum_hbm, deg_hbm, rbuf, cbuf, gbuf, obuf, acc, dacc):
        cid = jax.lax.axis_index("c")
        sid = jax.lax.axis_index("s")
        wid = cid * NS + sid

        @pl.loop(0, wb_per)
        def _(k):
            blk = sid + k * NS

            @pl.when(blk < nwb)
            def _():
                pltpu.sync_copy(zf_hbm, acc.at[pl.ds(blk * WB, WB)])
                pltpu.sync_copy(zd_hbm, dacc.at[pl.ds(blk * WB, WB)])

        pltpu.sync_copy(ones_hbm, obuf)
        plsc.subcore_barrier()

        base = wid * epw

        @pl.loop(0, chunks)
        def _(k):
            off = base + k * B
            pltpu.sync_copy(row_hbm.at[pl.ds(off, B)], rbuf)
            pltpu.sync_copy(col_hbm.at[pl.ds(off, B)], cbuf)
            pltpu.sync_copy(x_hbm.at[rbuf], gbuf)
            pltpu.sync_copy(gbuf, acc.at[cbuf], add=True)
            pltpu.sync_copy(obuf, dacc.at[cbuf], add=True)

        plsc.subcore_barrier()

        @pl.loop(0, wb_per)
        def _(k):
            blk = sid + k * NS

            @pl.when(blk < nwb)
            def _():
                pltpu.sync_copy(acc.at[pl.ds(blk * WB, WB)],
                                sum_hbm.at[pl.ds(cid * n + blk * WB, WB)])
                pltpu.sync_copy(dacc.at[pl.ds(blk * WB, WB)],
                                deg_hbm.at[pl.ds(cid * n + blk * WB, WB)])

    return agg_kernel(x, row, col, z_feat, z_deg, ones_rows)


def _tc_combine(x, psum, pdeg, wt, b2):
    n, d = x.shape
    dout = wt.shape[1]
    bm = 1000
    grid = n // bm

    def body(x_ref, p0_ref, p1_ref, d0_ref, d1_ref, wt_ref, b_ref, o_ref):
        deg = d0_ref[:, 0:1] + d1_ref[:, 0:1]
        deg = jnp.maximum(deg, 1.0)
        aggr = (p0_ref[...] + p1_ref[...]) / deg
        cat = jnp.concatenate([x_ref[...], aggr], axis=1)
        o_ref[...] = jnp.dot(cat, wt_ref[...],
                             preferred_element_type=jnp.float32) + b_ref[...]

    return pl.pallas_call(
        body,
        grid=(grid,),
        in_specs=[
            pl.BlockSpec((bm, d), lambda i: (i, 0)),
            pl.BlockSpec((bm, d), lambda i: (i, 0)),
            pl.BlockSpec((bm, d), lambda i, g=grid: (i + g, 0)),
            pl.BlockSpec((bm, LANES), lambda i: (i, 0)),
            pl.BlockSpec((bm, LANES), lambda i, g=grid: (i + g, 0)),
            pl.BlockSpec((2 * d, dout), lambda i: (0, 0)),
            pl.BlockSpec((1, dout), lambda i: (0, 0)),
        ],
        out_specs=pl.BlockSpec((bm, dout), lambda i: (i, 0)),
        out_shape=jax.ShapeDtypeStruct((n, dout), jnp.float32),
    )(x, psum, psum, pdeg, pdeg, wt, b2)


def kernel(x, edge_index, W, b):
    row = edge_index[0].astype(jnp.int32)
    col = edge_index[1].astype(jnp.int32)
    psum, pdeg = _sc_aggregate(x, row, col)
    return _tc_combine(x, psum, pdeg, W.T, b[None, :])


# speedup vs baseline: 5.6027x; 5.6027x over previous
import dataclasses
import functools

import jax
import jax.numpy as jnp
from jax.experimental import pallas as pl
from jax.experimental.pallas import tpu as pltpu
from jax.experimental.pallas import tpu_sc as plsc

NC = 2
NS = 16
LANES = 16
NW = NC * NS


def _sc_aggregate(x, row, col):
    n, d = x.shape
    e = row.shape[0]
    epw = e // NW
    B = 80
    chunks = epw // B
    WB = 80
    nwb = n // WB
    wb_per = (nwb + NS - 1) // NS

    mesh = plsc.VectorSubcoreMesh(
        core_axis_name="c", subcore_axis_name="s",
        num_cores=NC, num_subcores=NS)

    cp = pltpu.CompilerParams()
    if "needs_layout_passes" in pltpu.CompilerParams.__dataclass_fields__:
        cp = dataclasses.replace(cp, needs_layout_passes=False)

    z_feat = jnp.zeros((WB, d), jnp.float32)

    @functools.partial(
        pl.kernel,
        out_type=(jax.ShapeDtypeStruct((NC * n, d), jnp.float32),
                  jax.ShapeDtypeStruct((NW, n), jnp.float32)),
        mesh=mesh,
        compiler_params=cp,
        scratch_types=[
            pltpu.VMEM((B,), jnp.int32),
            pltpu.VMEM((B,), jnp.int32),
            pltpu.VMEM((B, d), jnp.float32),
            pltpu.VMEM((n,), jnp.float32),
            pltpu.VMEM_SHARED((n, d), jnp.float32),
        ],
    )
    def agg_kernel(x_hbm, row_hbm, col_hbm, zf_hbm,
                   sum_hbm, deg_hbm, rbuf, cbuf, gbuf, dloc, acc):
        cid = jax.lax.axis_index("c")
        sid = jax.lax.axis_index("s")
        wid = cid * NS + sid

        @pl.loop(0, wb_per)
        def _(k):
            blk = sid + k * NS

            @pl.when(blk < nwb)
            def _():
                pltpu.sync_copy(zf_hbm, acc.at[pl.ds(blk * WB, WB)])

        @pl.loop(0, n, step=LANES)
        def _(j):
            dloc[pl.ds(j, LANES)] = jnp.zeros((LANES,), jnp.float32)

        plsc.subcore_barrier()

        base = wid * epw
        ones_v = jnp.ones((LANES,), jnp.float32)

        @pl.loop(0, chunks)
        def _(k):
            off = base + k * B
            pltpu.sync_copy(row_hbm.at[pl.ds(off, B)], rbuf)
            pltpu.sync_copy(col_hbm.at[pl.ds(off, B)], cbuf)
            pltpu.sync_copy(x_hbm.at[rbuf], gbuf)
            pltpu.sync_copy(gbuf, acc.at[cbuf], add=True)

            @pl.loop(0, B, step=LANES)
            def _(j):
                idx = cbuf[pl.ds(j, LANES)]
                plsc.addupdate_scatter(dloc, [idx], ones_v)

        plsc.subcore_barrier()

        @pl.loop(0, wb_per)
        def _(k):
            blk = sid + k * NS

            @pl.when(blk < nwb)
            def _():
                pltpu.sync_copy(acc.at[pl.ds(blk * WB, WB)],
                                sum_hbm.at[pl.ds(cid * n + blk * WB, WB)])

        pltpu.sync_copy(dloc, deg_hbm.at[wid])

    return agg_kernel(x, row, col, z_feat)


def _tc_degsum(pdeg):
    n = pdeg.shape[1]
    ones_nw = jnp.ones((NW, 1), jnp.float32)

    def body(dg_ref, on_ref, o_ref):
        deg = jax.lax.dot_general(
            dg_ref[...], on_ref[...], (((0,), (0,)), ((), ())),
            preferred_element_type=jnp.float32)
        o_ref[...] = jnp.maximum(deg, 1.0)

    return pl.pallas_call(
        body,
        out_shape=jax.ShapeDtypeStruct((n, 1), jnp.float32),
    )(pdeg, ones_nw)


def _tc_combine(x, psum, deg, wt, b2):
    n, d = x.shape
    dout = wt.shape[1]
    bm = 1000
    grid = n // bm

    def body(x_ref, p0_ref, p1_ref, dg_ref, wt_ref, b_ref, o_ref):
        aggr = (p0_ref[...] + p1_ref[...]) / dg_ref[...]
        cat = jnp.concatenate([x_ref[...], aggr], axis=1)
        o_ref[...] = jnp.dot(cat, wt_ref[...],
                             preferred_element_type=jnp.float32) + b_ref[...]

    return pl.pallas_call(
        body,
        grid=(grid,),
        in_specs=[
            pl.BlockSpec((bm, d), lambda i: (i, 0)),
            pl.BlockSpec((bm, d), lambda i: (i, 0)),
            pl.BlockSpec((bm, d), lambda i, g=grid: (i + g, 0)),
            pl.BlockSpec((bm, 1), lambda i: (i, 0)),
            pl.BlockSpec((2 * d, dout), lambda i: (0, 0)),
            pl.BlockSpec((1, dout), lambda i: (0, 0)),
        ],
        out_specs=pl.BlockSpec((bm, dout), lambda i: (i, 0)),
        out_shape=jax.ShapeDtypeStruct((n, dout), jnp.float32),
    )(x, psum, psum, deg, wt, b2)


def kernel(x, edge_index, W, b):
    row = edge_index[0].astype(jnp.int32)
    col = edge_index[1].astype(jnp.int32)
    psum, pdeg = _sc_aggregate(x, row, col)
    deg = _tc_degsum(pdeg)
    return _tc_combine(x, psum, deg, W.T, b[None, :])
